# baseline (speedup 1.0000x reference)
import functools

import jax
import jax.numpy as jnp
from jax import lax
from jax.experimental import pallas as pl
from jax.experimental.pallas import tpu as pltpu, tpu_sc as plsc

D = 128
W = D // 2
C = 256

_info = plsc.get_sparse_core_info()
NC, NS, L = _info.num_cores, _info.num_subcores, _info.num_lanes
NW = NC * NS


def _make_sc_kernel(padded_b: int):
    nch = padded_b // (NW * C)
    assert nch % 2 == 0
    mesh = plsc.VectorSubcoreMesh(core_axis_name="c", subcore_axis_name="s")

    @functools.partial(
        pl.kernel,
        mesh=mesh,
        compiler_params=pltpu.CompilerParams(
            needs_layout_passes=False, use_tc_tiling_on_sc=False),
        out_type=jax.ShapeDtypeStruct((padded_b,), jnp.float32),
        scratch_types=[
            pltpu.VMEM((2 * nch, C), jnp.int32),
            pltpu.VMEM((C, W), jnp.float32),
            pltpu.VMEM((C, W), jnp.float32),
            pltpu.VMEM((C, W), jnp.float32),
            pltpu.VMEM((C, W), jnp.float32),
            pltpu.VMEM((C,), jnp.float32),
            pltpu.VMEM((C,), jnp.float32),
            pltpu.SemaphoreType.DMA,
            pltpu.SemaphoreType.DMA,
            pltpu.SemaphoreType.DMA,
            pltpu.SemaphoreType.DMA,
        ],
    )
    def sc_kernel(u_tbl, m_tbl, idx_hbm, out_hbm,
                  idxv, u0, m0, u1, m1, o0, o1, gs0, gs1, os0, os1):
        wid = lax.axis_index("s") * NC + lax.axis_index("c")

        pltpu.sync_copy(idx_hbm.at[wid], idxv)

        def fire(i, ub, mb, gs):
            pltpu.async_copy(u_tbl.at[idxv.at[2 * i]], ub, gs)
            pltpu.async_copy(m_tbl.at[idxv.at[2 * i + 1]], mb, gs)

        def drain(i, ub, mb, gs):
            pltpu.make_async_copy(u_tbl.at[idxv.at[2 * i]], ub, gs).wait()
            pltpu.make_async_copy(m_tbl.at[idxv.at[2 * i + 1]], mb, gs).wait()

        def out_slice(i):
            return out_hbm.at[pl.ds((i * NW + wid) * C, C)]

        lanes = lax.iota(jnp.int32, L)
        last_lane = lanes == (L - 1)
        zeros_i = jnp.zeros((L,), jnp.int32)

        def compute(ub, mb, ob):
            @plsc.parallel_loop(0, C, unroll=4)
            def _(e):
                acc = jnp.zeros((L,), jnp.float32)
                for w in range(W // L):
                    uw = ub[e, pl.ds(w * L, L)]
                    mw = mb[e, pl.ds(w * L, L)]
                    p = (plsc.bitcast(uw, jnp.bfloat16)
                         * plsc.bitcast(mw, jnp.bfloat16))
                    pa, pb = plsc.unpack(
                        p, format=plsc.PackFormat.INTERLEAVED,
                        preferred_element_type=jnp.float32)
                    acc = acc + (pa + pb)
                cs = plsc.cumsum(acc)
                plsc.store_scatter(ob, [zeros_i + e], cs, mask=last_lane)

        fire(0, u0, m0, gs0)

        def pair_body(k, _):
            i = 2 * k
            fire(i + 1, u1, m1, gs1)
            drain(i, u0, m0, gs0)

            @pl.when(k > 0)
            def _():
                pltpu.make_async_copy(o0, out_slice(i - 2), os0).wait()

            compute(u0, m0, o0)
            pltpu.async_copy(o0, out_slice(i), os0)

            @pl.when(k < nch // 2 - 1)
            def _():
                fire(i + 2, u0, m0, gs0)

            drain(i + 1, u1, m1, gs1)

            @pl.when(k > 0)
            def _():
                pltpu.make_async_copy(o1, out_slice(i - 1), os1).wait()

            compute(u1, m1, o1)
            pltpu.async_copy(o1, out_slice(i + 1), os1)
            return ()

        lax.fori_loop(0, nch // 2, pair_body, ())

        pltpu.make_async_copy(o0, out_slice(nch - 2), os0).wait()
        pltpu.make_async_copy(o1, out_slice(nch - 1), os1).wait()

    return sc_kernel


def _pack_table(x):
    x16 = x.astype(jnp.bfloat16)
    return jax.lax.bitcast_convert_type(
        x16.reshape(x.shape[0], W, 2), jnp.float32)


def kernel(x_user, x_movie, edge_label_index):
    eli = edge_label_index.astype(jnp.int32)
    b = eli.shape[1]
    grain = 2 * NW * C
    padded_b = ((b + grain - 1) // grain) * grain
    nch = padded_b // (NW * C)
    uidx = jnp.pad(eli[0], (0, padded_b - b))
    midx = jnp.pad(eli[1], (0, padded_b - b))
    ur = uidx.reshape(nch, NW, C)
    mr = midx.reshape(nch, NW, C)
    idx = jnp.stack([ur, mr], axis=1)
    idx = idx.transpose(2, 0, 1, 3).reshape(NW, 2 * nch, C)
    out = _make_sc_kernel(padded_b)(
        _pack_table(x_user), _pack_table(x_movie), idx)
    return out[:b]


# speedup vs baseline: 1.0413x; 1.0413x over previous
import functools

import jax
import jax.numpy as jnp
from jax import lax
from jax.experimental import pallas as pl
from jax.experimental.pallas import tpu as pltpu, tpu_sc as plsc

D = 128
W = D // 2
C = 128

_info = plsc.get_sparse_core_info()
NC, NS, L = _info.num_cores, _info.num_subcores, _info.num_lanes
NW = NC * NS


def _make_sc_kernel(padded_b: int):
    nch = padded_b // (NW * C)
    assert nch % 2 == 0
    mesh = plsc.VectorSubcoreMesh(core_axis_name="c", subcore_axis_name="s")

    @functools.partial(
        pl.kernel,
        mesh=mesh,
        compiler_params=pltpu.CompilerParams(
            needs_layout_passes=False, use_tc_tiling_on_sc=False),
        out_type=jax.ShapeDtypeStruct((padded_b,), jnp.float32),
        scratch_types=[
            pltpu.VMEM((2 * nch, C), jnp.int32),
            pltpu.VMEM((C, W), jnp.float32),
            pltpu.VMEM((C, W), jnp.float32),
            pltpu.VMEM((C, W), jnp.float32),
            pltpu.VMEM((C, W), jnp.float32),
            pltpu.VMEM((C,), jnp.float32),
            pltpu.VMEM((C,), jnp.float32),
            pltpu.SemaphoreType.DMA,
            pltpu.SemaphoreType.DMA,
            pltpu.SemaphoreType.DMA,
            pltpu.SemaphoreType.DMA,
        ],
    )
    def sc_kernel(u_tbl, m_tbl, idx_hbm, out_hbm,
                  idxv, u0, m0, u1, m1, o0, o1, gs0, gs1, os0, os1):
        wid = lax.axis_index("s") * NC + lax.axis_index("c")

        pltpu.sync_copy(idx_hbm.at[wid], idxv)

        def fire(i, ub, mb, gs):
            pltpu.async_copy(u_tbl.at[idxv.at[2 * i, pl.ds(0, C // 2)]],
                             ub.at[pl.ds(0, C // 2)], gs)
            pltpu.async_copy(m_tbl.at[idxv.at[2 * i + 1, pl.ds(0, C // 2)]],
                             mb.at[pl.ds(0, C // 2)], gs)
            pltpu.async_copy(u_tbl.at[idxv.at[2 * i, pl.ds(C // 2, C // 2)]],
                             ub.at[pl.ds(C // 2, C // 2)], gs)
            pltpu.async_copy(m_tbl.at[idxv.at[2 * i + 1, pl.ds(C // 2, C // 2)]],
                             mb.at[pl.ds(C // 2, C // 2)], gs)

        def drain(i, ub, mb, gs):
            pltpu.make_async_copy(u_tbl.at[idxv.at[2 * i, pl.ds(0, C // 2)]],
                                  ub.at[pl.ds(0, C // 2)], gs).wait()
            pltpu.make_async_copy(m_tbl.at[idxv.at[2 * i + 1, pl.ds(0, C // 2)]],
                                  mb.at[pl.ds(0, C // 2)], gs).wait()
            pltpu.make_async_copy(u_tbl.at[idxv.at[2 * i, pl.ds(C // 2, C // 2)]],
                                  ub.at[pl.ds(C // 2, C // 2)], gs).wait()
            pltpu.make_async_copy(m_tbl.at[idxv.at[2 * i + 1, pl.ds(C // 2, C // 2)]],
                                  mb.at[pl.ds(C // 2, C // 2)], gs).wait()

        def out_slice(i):
            return out_hbm.at[pl.ds((i * NW + wid) * C, C)]

        lanes = lax.iota(jnp.int32, L)
        last_lane = lanes == (L - 1)
        zeros_i = jnp.zeros((L,), jnp.int32)

        def compute(ub, mb, ob):
            @plsc.parallel_loop(0, C, unroll=4)
            def _(e):
                acc = jnp.zeros((L,), jnp.float32)
                for w in range(W // L):
                    uw = ub[e, pl.ds(w * L, L)]
                    mw = mb[e, pl.ds(w * L, L)]
                    p = (plsc.bitcast(uw, jnp.bfloat16)
                         * plsc.bitcast(mw, jnp.bfloat16))
                    pa, pb = plsc.unpack(
                        p, format=plsc.PackFormat.INTERLEAVED,
                        preferred_element_type=jnp.float32)
                    acc = acc + (pa + pb)
                cs = plsc.cumsum(acc)
                plsc.store_scatter(ob, [zeros_i + e], cs, mask=last_lane)

        fire(0, u0, m0, gs0)

        def pair_body(k, _):
            i = 2 * k
            fire(i + 1, u1, m1, gs1)
            drain(i, u0, m0, gs0)

            @pl.when(k > 0)
            def _():
                pltpu.make_async_copy(o0, out_slice(i - 2), os0).wait()

            compute(u0, m0, o0)
            pltpu.async_copy(o0, out_slice(i), os0)

            @pl.when(k < nch // 2 - 1)
            def _():
                fire(i + 2, u0, m0, gs0)

            drain(i + 1, u1, m1, gs1)

            @pl.when(k > 0)
            def _():
                pltpu.make_async_copy(o1, out_slice(i - 1), os1).wait()

            compute(u1, m1, o1)
            pltpu.async_copy(o1, out_slice(i + 1), os1)
            return ()

        lax.fori_loop(0, nch // 2, pair_body, ())

        pltpu.make_async_copy(o0, out_slice(nch - 2), os0).wait()
        pltpu.make_async_copy(o1, out_slice(nch - 1), os1).wait()

    return sc_kernel


def _pack_table(x):
    x16 = x.astype(jnp.bfloat16)
    return jax.lax.bitcast_convert_type(
        x16.reshape(x.shape[0], W, 2), jnp.float32)


def kernel(x_user, x_movie, edge_label_index):
    eli = edge_label_index.astype(jnp.int32)
    b = eli.shape[1]
    grain = 2 * NW * C
    padded_b = ((b + grain - 1) // grain) * grain
    nch = padded_b // (NW * C)
    uidx = jnp.pad(eli[0], (0, padded_b - b))
    midx = jnp.pad(eli[1], (0, padded_b - b))
    ur = uidx.reshape(nch, NW, C)
    mr = midx.reshape(nch, NW, C)
    idx = jnp.stack([ur, mr], axis=1)
    idx = idx.transpose(2, 0, 1, 3).reshape(NW, 2 * nch, C)
    out = _make_sc_kernel(padded_b)(
        _pack_table(x_user), _pack_table(x_movie), idx)
    return out[:b]


# speedup vs baseline: 1.5920x; 1.5288x over previous
import functools

import jax
import jax.numpy as jnp
from jax import lax
from jax.experimental import pallas as pl
from jax.experimental.pallas import tpu as pltpu, tpu_sc as plsc

D = 128
W = D // 2
C = 100

_info = plsc.get_sparse_core_info()
NC, NS, L = _info.num_cores, _info.num_subcores, _info.num_lanes
NW = NC * NS


def _make_sc_kernel(padded_b: int):
    nch = padded_b // (NW * C)
    assert nch % 2 == 0
    mesh = plsc.VectorSubcoreMesh(core_axis_name="c", subcore_axis_name="s")

    @functools.partial(
        pl.kernel,
        mesh=mesh,
        compiler_params=pltpu.CompilerParams(
            needs_layout_passes=False, use_tc_tiling_on_sc=False),
        out_type=jax.ShapeDtypeStruct((padded_b // C, C), jnp.float32),
        scratch_types=[
            pltpu.VMEM((2 * nch, C), jnp.int32),
            pltpu.VMEM((C, W), jnp.float32),
            pltpu.VMEM((C, W), jnp.float32),
            pltpu.VMEM((C, W), jnp.float32),
            pltpu.VMEM((C, W), jnp.float32),
            pltpu.VMEM((C,), jnp.float32),
            pltpu.VMEM((C,), jnp.float32),
            pltpu.SemaphoreType.DMA,
            pltpu.SemaphoreType.DMA,
            pltpu.SemaphoreType.DMA,
            pltpu.SemaphoreType.DMA,
        ],
    )
    def sc_kernel(u_tbl, m_tbl, idx_hbm, out_hbm,
                  idxv, u0, m0, u1, m1, o0, o1, gs0, gs1, os0, os1):
        wid = lax.axis_index("s") * NC + lax.axis_index("c")

        pltpu.sync_copy(idx_hbm.at[wid], idxv)

        def fire(i, ub, mb, gs):
            pltpu.async_copy(u_tbl.at[idxv.at[2 * i]], ub, gs)
            pltpu.async_copy(m_tbl.at[idxv.at[2 * i + 1]], mb, gs)

        def drain(i, ub, mb, gs):
            pltpu.make_async_copy(u_tbl.at[idxv.at[2 * i]], ub, gs).wait()
            pltpu.make_async_copy(m_tbl.at[idxv.at[2 * i + 1]], mb, gs).wait()

        def out_slice(i):
            return out_hbm.at[i * NW + wid]

        lanes = lax.iota(jnp.int32, L)
        last_lane = lanes == (L - 1)
        zeros_i = jnp.zeros((L,), jnp.int32)

        def compute(ub, mb, ob):
            @plsc.parallel_loop(0, C, unroll=4)
            def _(e):
                acc = jnp.zeros((L,), jnp.float32)
                for w in range(W // L):
                    uw = ub[e, pl.ds(w * L, L)]
                    mw = mb[e, pl.ds(w * L, L)]
                    p = (plsc.bitcast(uw, jnp.bfloat16)
                         * plsc.bitcast(mw, jnp.bfloat16))
                    pa, pb = plsc.unpack(
                        p, format=plsc.PackFormat.INTERLEAVED,
                        preferred_element_type=jnp.float32)
                    acc = acc + (pa + pb)
                cs = plsc.cumsum(acc)
                plsc.store_scatter(ob, [zeros_i + e], cs, mask=last_lane)

        fire(0, u0, m0, gs0)

        def pair_body(k, _):
            i = 2 * k
            fire(i + 1, u1, m1, gs1)
            drain(i, u0, m0, gs0)

            @pl.when(k > 0)
            def _():
                pltpu.make_async_copy(o0, out_slice(i - 2), os0).wait()

            compute(u0, m0, o0)
            pltpu.async_copy(o0, out_slice(i), os0)

            @pl.when(k < nch // 2 - 1)
            def _():
                fire(i + 2, u0, m0, gs0)

            drain(i + 1, u1, m1, gs1)

            @pl.when(k > 0)
            def _():
                pltpu.make_async_copy(o1, out_slice(i - 1), os1).wait()

            compute(u1, m1, o1)
            pltpu.async_copy(o1, out_slice(i + 1), os1)
            return ()

        lax.fori_loop(0, nch // 2, pair_body, ())

        pltpu.make_async_copy(o0, out_slice(nch - 2), os0).wait()
        pltpu.make_async_copy(o1, out_slice(nch - 1), os1).wait()

    return sc_kernel


def _pack_table(x):
    x16 = x.astype(jnp.bfloat16)
    return jax.lax.bitcast_convert_type(
        x16.reshape(x.shape[0], W, 2), jnp.float32)


def kernel(x_user, x_movie, edge_label_index):
    eli = edge_label_index.astype(jnp.int32)
    b = eli.shape[1]
    grain = 2 * NW * C
    padded_b = ((b + grain - 1) // grain) * grain
    nch = padded_b // (NW * C)
    uidx = jnp.pad(eli[0], (0, padded_b - b))
    midx = jnp.pad(eli[1], (0, padded_b - b))
    ur = uidx.reshape(nch, NW, C)
    mr = midx.reshape(nch, NW, C)
    idx = jnp.stack([ur, mr], axis=1)
    idx = idx.transpose(2, 0, 1, 3).reshape(NW, 2 * nch, C)
    out = _make_sc_kernel(padded_b)(
        _pack_table(x_user), _pack_table(x_movie), idx)
    return out.reshape(padded_b)[:b]
